# baseline (speedup 1.0000x reference)
import functools

import jax
import jax.numpy as jnp
from jax.experimental import pallas as pl
from jax.experimental.pallas import tpu as pltpu

N = 10000
E = 200
NF = 128
EF = 30
D = 128
TD = 16

NT = 10
TILE = N // NT


def _dg(a, b):
    return jax.lax.dot_general(a, b, (((1,), (1,)), ((), ())),
                               preferred_element_type=jnp.float32)


def _dgT(a, b):
    return jax.lax.dot_general(a, b, (((0,), (0,)), ((), ())),
                               preferred_element_type=jnp.float32)


def _tgn_kernel(src_ref, dst_ref,
                nf_ref, ts_ref, ef_ref,
                srcc_ref, dstc_ref, allr_ref,
                Wnp_ref, bnp_ref,
                w0_ref, b0_ref, tw_ref, tb_ref,
                Wmsg_ref, Wtl_ref, bmsg_ref,
                Wih_ref, bih_ref,
                Whh_ref, bhh_ref,
                Wgate_ref, bgate_ref,
                Wproj_ref, bproj_ref,
                out_ref,
                mem_ref, econst_ref, T_ref):
    mem_ref[...] = _dg(nf_ref[...], Wnp_ref[...]).astype(jnp.float32) + bnp_ref[...]

    if True:
        t = ts_ref[...]
        lin = t * w0_ref[0, 0] + b0_ref[0, 0]
        sn = jnp.sin(t * tw_ref[...] + tb_ref[...])
        Wmsg = Wmsg_ref[...]
        W_e = Wmsg[:, 2 * D:2 * D + EF]
        W_ts = Wmsg[:, 2 * D + EF + 1:]
        econst_ref[...] = (_dg(ef_ref[...], W_e) + lin * Wtl_ref[...]
                           + _dg(sn, W_ts) + bmsg_ref[...])

        W_sd = Wmsg[:, :2 * D]
        Wih = Wih_ref[...]
        bih = bih_ref[...]
        Whh = Whh_ref[...]
        bhh = bhh_ref[...]
        econst = econst_ref[...]

        def init_body(e, carry):
            s = src_ref[e]
            d = dst_ref[e]
            T_ref[pl.ds(e, 1), :] = mem_ref[pl.ds(s, 1), :]
            T_ref[pl.ds(E + e, 1), :] = mem_ref[pl.ds(d, 1), :]
            return carry

        jax.lax.fori_loop(0, E, init_body, 0, unroll=8)

        src_c = srcc_ref[...]
        dst_c = dstc_ref[...]
        all_r = allr_ref[...]
        src_r = all_r[:, :E]
        dst_r = all_r[:, E:]

        eqs = (src_c == all_r).astype(jnp.float32)
        eqd = (dst_c == all_r).astype(jnp.float32)
        bsm = eqs * (1.0 - eqd)
        eqsd = (src_c == dst_c)

        conf = ((src_c == src_r) | (src_c == dst_r)
                | (dst_c == src_r) | (dst_c == dst_r))
        row_i = jax.lax.broadcasted_iota(jnp.int32, (E, E), 0)
        col_i = jax.lax.broadcasted_iota(jnp.int32, (E, E), 1)
        lower = col_i < row_i
        CL = (conf & lower).astype(jnp.float32)
        ident = (row_i == col_i).astype(jnp.float32)
        ones8 = jnp.ones((E, 8), jnp.float32)

        def gru_combine(gi, gh, h):
            r = jax.nn.sigmoid(gi[:, :D] + gh[:, :D])
            z = jax.nn.sigmoid(gi[:, D:2 * D] + gh[:, D:2 * D])
            n = jnp.tanh(gi[:, 2 * D:] + r * gh[:, 2 * D:])
            return (1.0 - z) * n + z * h

        def wave_cond(carry):
            com_c, com_r = carry
            return jnp.sum(com_c) < jnp.float32(E)

        def wave_body(carry):
            com_c, com_r = carry
            blocked = jnp.max(CL * (1.0 - com_r), axis=1, keepdims=True)
            active = (1.0 - com_c) * (1.0 - blocked)

            Tv = T_ref[...]
            s_rows = Tv[:E, :]
            d_rows = Tv[E:, :]
            sd_flat = jnp.concatenate([s_rows, d_rows], axis=1)
            pre = _dg(sd_flat, W_sd) + econst
            msg = jnp.maximum(pre, 0.0)
            gh_all = _dg(Tv, Whh) + bhh
            gi = _dg(msg, Wih) + bih
            upd_s = gru_combine(gi, gh_all[:E, :], s_rows)
            gh_d2 = _dg(upd_s, Whh) + bhh
            gh_d = jnp.where(eqsd, gh_d2, gh_all[E:, :])
            h2 = jnp.where(eqsd, upd_s, d_rows)
            upd_d = gru_combine(gi, gh_d, h2)

            A_s = bsm * active
            A_d = eqd * active
            sc_s = _dgT(A_s, upd_s)
            sc_d = _dgT(A_d, upd_d)
            cov = _dgT(A_s + A_d, ones8)[:, :1]
            T_ref[...] = Tv * (1.0 - cov) + sc_s + sc_d

            com_c = com_c + active
            com8 = jnp.broadcast_to(com_c, (E, 8))
            com_r = _dgT(com8, ident)[:1, :]
            return com_c, com_r

        jax.lax.while_loop(
            wave_cond, wave_body,
            (jnp.zeros((E, 1), jnp.float32), jnp.zeros((1, E), jnp.float32)))

        def fin_body(e, carry):
            s = src_ref[e]
            d = dst_ref[e]
            mem_ref[pl.ds(s, 1), :] = T_ref[pl.ds(e, 1), :]
            mem_ref[pl.ds(d, 1), :] = T_ref[pl.ds(E + e, 1), :]
            return carry

        jax.lax.fori_loop(0, E, fin_body, 0, unroll=8)

        Wgate = Wgate_ref[...]
        bgate = bgate_ref[0, 0]
        Wproj = Wproj_ref[...]

        def ro_body(k, carry):
            m, zz, acc = carry
            tile = mem_ref[pl.ds(k * TILE, TILE), :]
            g = jnp.sum(tile * Wgate, axis=1, keepdims=True) + bgate
            mt = jnp.maximum(m, jnp.max(g))
            scale = jnp.exp(m - mt)
            w = jnp.exp(g - mt)
            p = _dg(tile, Wproj)
            acc = acc * scale + jnp.sum(w * p, axis=0, keepdims=True)
            zz = zz * scale + jnp.sum(w)
            return mt, zz, acc

        m, zz, acc = jax.lax.fori_loop(
            0, NT, ro_body,
            (jnp.float32(-jnp.inf), jnp.float32(0.0),
             jnp.zeros((1, D), jnp.float32)))
        out_ref[...] = acc / zz + bproj_ref[...]


@functools.partial(jax.jit, static_argnames=("interpret",))
def kernel(node_features, timestamps, edge_features, W_np, b_np, t2v_w0,
           t2v_b0, t2v_w, t2v_b, W_msg, b_msg, W_ih, b_ih, W_hh, b_hh,
           W_gate, b_gate, W_proj, b_proj, sources, destinations,
           interpret=False):
    src = sources.astype(jnp.int32)
    dst = destinations.astype(jnp.int32)
    ts = timestamps.reshape(E, 1).astype(jnp.float32)
    src_col = src.reshape(E, 1)
    dst_col = dst.reshape(E, 1)
    all_row = jnp.concatenate([src, dst]).reshape(1, 2 * E)

    smem = lambda: pl.BlockSpec(memory_space=pltpu.SMEM)
    vfull = lambda: pl.BlockSpec(memory_space=pltpu.VMEM)

    grid_spec = pltpu.PrefetchScalarGridSpec(
        num_scalar_prefetch=2,
        grid=(1,),
        in_specs=[
            vfull(),
            vfull(),
            vfull(),
            vfull(), vfull(), vfull(),
            vfull(), vfull(),
            smem(), smem(),
            vfull(), vfull(),
            vfull(), vfull(), vfull(),
            vfull(), vfull(),
            vfull(), vfull(),
            vfull(), smem(),
            vfull(), vfull(),
        ],
        out_specs=pl.BlockSpec((1, D), lambda i, *_: (0, 0)),
        scratch_shapes=[
            pltpu.VMEM((N, D), jnp.float32),
            pltpu.VMEM((E, D), jnp.float32),
            pltpu.VMEM((2 * E, D), jnp.float32),
        ],
    )

    pooled = pl.pallas_call(
        _tgn_kernel,
        grid_spec=grid_spec,
        out_shape=jax.ShapeDtypeStruct((1, D), jnp.float32),
        compiler_params=pltpu.CompilerParams(
            dimension_semantics=("arbitrary",)),
        interpret=interpret,
    )(src, dst,
      node_features.astype(jnp.bfloat16), ts, edge_features,
      src_col, dst_col, all_row,
      W_np.astype(jnp.bfloat16), b_np.reshape(1, D),
      t2v_w0.reshape(1, 1), t2v_b0.reshape(1, 1),
      t2v_w.reshape(1, TD - 1), t2v_b.reshape(1, TD - 1),
      W_msg, W_msg[:, 2 * D + EF:2 * D + EF + 1].T, b_msg.reshape(1, D),
      W_ih, b_ih.reshape(1, 3 * D),
      W_hh, b_hh.reshape(1, 3 * D),
      W_gate, b_gate.reshape(1, 1),
      W_proj, b_proj.reshape(1, D))
    return pooled.reshape(D)


# speedup vs baseline: 1.1674x; 1.1674x over previous
import functools

import jax
import jax.numpy as jnp
from jax.experimental import pallas as pl
from jax.experimental.pallas import tpu as pltpu

N = 10000
E = 200
NF = 128
EF = 30
D = 128
TD = 16

NT = 10
TILE = N // NT


def _dg(a, b):
    return jax.lax.dot_general(a, b, (((1,), (1,)), ((), ())),
                               preferred_element_type=jnp.float32)


def _dgT(a, b):
    return jax.lax.dot_general(a, b, (((0,), (0,)), ((), ())),
                               preferred_element_type=jnp.float32)


def _tgn_kernel(src_ref, dst_ref,
                nf_ref, ts_ref, ef_ref,
                srcc_ref, dstc_ref, allr_ref,
                Wnp_ref, bnp_ref,
                w0_ref, b0_ref, tw_ref, tb_ref,
                Wmsg_ref, Wtl_ref, bmsg_ref,
                Wih_ref, bih_ref,
                Whh_ref, bhh_ref,
                Wgate_ref, bgate_ref,
                Wproj_ref, bproj_ref,
                out_ref,
                mem_ref, econst_ref, T_ref):
    mem_ref[...] = _dg(nf_ref[...], Wnp_ref[...]) + bnp_ref[...]

    if True:
        t = ts_ref[...]
        lin = t * w0_ref[0, 0] + b0_ref[0, 0]
        sn = jnp.sin(t * tw_ref[...] + tb_ref[...])
        Wmsg = Wmsg_ref[...]
        W_e = Wmsg[:, 2 * D:2 * D + EF]
        W_ts = Wmsg[:, 2 * D + EF + 1:]
        econst_ref[...] = (_dg(ef_ref[...], W_e) + lin * Wtl_ref[...]
                           + _dg(sn, W_ts) + bmsg_ref[...])

        W_sd = Wmsg[:, :2 * D]
        Wih = Wih_ref[...]
        bih = bih_ref[...]
        Whh = Whh_ref[...]
        bhh = bhh_ref[...]
        econst = econst_ref[...]

        def init_body(e, carry):
            s = src_ref[e]
            d = dst_ref[e]
            T_ref[pl.ds(e, 1), :] = mem_ref[pl.ds(s, 1), :]
            T_ref[pl.ds(E + e, 1), :] = mem_ref[pl.ds(d, 1), :]
            return carry

        jax.lax.fori_loop(0, E, init_body, 0, unroll=8)

        src_c = srcc_ref[...]
        dst_c = dstc_ref[...]
        all_r = allr_ref[...]
        src_r = all_r[:, :E]
        dst_r = all_r[:, E:]

        eqs = (src_c == all_r).astype(jnp.float32)
        eqd = (dst_c == all_r).astype(jnp.float32)
        bsm = eqs * (1.0 - eqd)
        eqsd = (src_c == dst_c)

        conf = ((src_c == src_r) | (src_c == dst_r)
                | (dst_c == src_r) | (dst_c == dst_r))
        row_i = jax.lax.broadcasted_iota(jnp.int32, (E, E), 0)
        col_i = jax.lax.broadcasted_iota(jnp.int32, (E, E), 1)
        lower = col_i < row_i
        CL = (conf & lower).astype(jnp.float32)
        ident = (row_i == col_i).astype(jnp.float32)
        ones8 = jnp.ones((E, 8), jnp.float32)

        def gru_combine(gi, gh, h):
            r = jax.nn.sigmoid(gi[:, :D] + gh[:, :D])
            z = jax.nn.sigmoid(gi[:, D:2 * D] + gh[:, D:2 * D])
            n = jnp.tanh(gi[:, 2 * D:] + r * gh[:, 2 * D:])
            return (1.0 - z) * n + z * h

        def wave_cond(carry):
            com_c, com_r = carry
            return jnp.sum(com_c) < jnp.float32(E)

        def wave_body(carry):
            com_c, com_r = carry
            blocked = jnp.max(CL * (1.0 - com_r), axis=1, keepdims=True)
            active = (1.0 - com_c) * (1.0 - blocked)

            Tv = T_ref[...]
            s_rows = Tv[:E, :]
            d_rows = Tv[E:, :]
            sd_flat = jnp.concatenate([s_rows, d_rows], axis=1)
            pre = _dg(sd_flat, W_sd) + econst
            msg = jnp.maximum(pre, 0.0)
            gh_all = _dg(Tv, Whh) + bhh
            gi = _dg(msg, Wih) + bih
            upd_s = gru_combine(gi, gh_all[:E, :], s_rows)
            gh_d2 = _dg(upd_s, Whh) + bhh
            gh_d = jnp.where(eqsd, gh_d2, gh_all[E:, :])
            h2 = jnp.where(eqsd, upd_s, d_rows)
            upd_d = gru_combine(gi, gh_d, h2)

            A_s = bsm * active
            A_d = eqd * active
            sc_s = _dgT(A_s, upd_s)
            sc_d = _dgT(A_d, upd_d)
            cov = _dgT(A_s + A_d, ones8)[:, :1]
            T_ref[...] = Tv * (1.0 - cov) + sc_s + sc_d

            com_c = com_c + active
            com8 = jnp.broadcast_to(com_c, (E, 8))
            com_r = _dgT(com8, ident)[:1, :]
            return com_c, com_r

        jax.lax.while_loop(
            wave_cond, wave_body,
            (jnp.zeros((E, 1), jnp.float32), jnp.zeros((1, E), jnp.float32)))

        def fin_body(e, carry):
            s = src_ref[e]
            d = dst_ref[e]
            mem_ref[pl.ds(s, 1), :] = T_ref[pl.ds(e, 1), :]
            mem_ref[pl.ds(d, 1), :] = T_ref[pl.ds(E + e, 1), :]
            return carry

        jax.lax.fori_loop(0, E, fin_body, 0, unroll=8)

        Wgate = Wgate_ref[...]
        bgate = bgate_ref[0, 0]
        Wproj = Wproj_ref[...]

        def ro_body(k, carry):
            m, zz, acc = carry
            tile = mem_ref[pl.ds(k * TILE, TILE), :]
            g = jnp.sum(tile * Wgate, axis=1, keepdims=True) + bgate
            mt = jnp.maximum(m, jnp.max(g))
            scale = jnp.exp(m - mt)
            w = jnp.exp(g - mt)
            p = _dg(tile, Wproj)
            acc = acc * scale + jnp.sum(w * p, axis=0, keepdims=True)
            zz = zz * scale + jnp.sum(w)
            return mt, zz, acc

        m, zz, acc = jax.lax.fori_loop(
            0, NT, ro_body,
            (jnp.float32(-jnp.inf), jnp.float32(0.0),
             jnp.zeros((1, D), jnp.float32)))
        out_ref[...] = acc / zz + bproj_ref[...]


@functools.partial(jax.jit, static_argnames=("interpret",))
def kernel(node_features, timestamps, edge_features, W_np, b_np, t2v_w0,
           t2v_b0, t2v_w, t2v_b, W_msg, b_msg, W_ih, b_ih, W_hh, b_hh,
           W_gate, b_gate, W_proj, b_proj, sources, destinations,
           interpret=False):
    src = sources.astype(jnp.int32)
    dst = destinations.astype(jnp.int32)
    ts = timestamps.reshape(E, 1).astype(jnp.float32)
    src_col = src.reshape(E, 1)
    dst_col = dst.reshape(E, 1)
    all_row = jnp.concatenate([src, dst]).reshape(1, 2 * E)

    smem = lambda: pl.BlockSpec(memory_space=pltpu.SMEM)
    vfull = lambda: pl.BlockSpec(memory_space=pltpu.VMEM)

    grid_spec = pltpu.PrefetchScalarGridSpec(
        num_scalar_prefetch=2,
        grid=(1,),
        in_specs=[
            vfull(),
            vfull(),
            vfull(),
            vfull(), vfull(), vfull(),
            vfull(), vfull(),
            smem(), smem(),
            vfull(), vfull(),
            vfull(), vfull(), vfull(),
            vfull(), vfull(),
            vfull(), vfull(),
            vfull(), smem(),
            vfull(), vfull(),
        ],
        out_specs=pl.BlockSpec((1, D), lambda i, *_: (0, 0)),
        scratch_shapes=[
            pltpu.VMEM((N, D), jnp.float32),
            pltpu.VMEM((E, D), jnp.float32),
            pltpu.VMEM((2 * E, D), jnp.float32),
        ],
    )

    pooled = pl.pallas_call(
        _tgn_kernel,
        grid_spec=grid_spec,
        out_shape=jax.ShapeDtypeStruct((1, D), jnp.float32),
        compiler_params=pltpu.CompilerParams(
            dimension_semantics=("arbitrary",)),
        interpret=interpret,
    )(src, dst,
      node_features, ts, edge_features,
      src_col, dst_col, all_row,
      W_np, b_np.reshape(1, D),
      t2v_w0.reshape(1, 1), t2v_b0.reshape(1, 1),
      t2v_w.reshape(1, TD - 1), t2v_b.reshape(1, TD - 1),
      W_msg, W_msg[:, 2 * D + EF:2 * D + EF + 1].T, b_msg.reshape(1, D),
      W_ih, b_ih.reshape(1, 3 * D),
      W_hh, b_hh.reshape(1, 3 * D),
      W_gate, b_gate.reshape(1, 1),
      W_proj, b_proj.reshape(1, D))
    return pooled.reshape(D)


# speedup vs baseline: 1.2796x; 1.0961x over previous
import functools

import jax
import jax.numpy as jnp
from jax.experimental import pallas as pl
from jax.experimental.pallas import tpu as pltpu

N = 10000
E = 200
NF = 128
EF = 30
D = 128
TD = 16

NT = 10
TILE = N // NT


def _dg(a, b):
    return jax.lax.dot_general(a, b, (((1,), (1,)), ((), ())),
                               preferred_element_type=jnp.float32)


def _dgT(a, b):
    return jax.lax.dot_general(a, b, (((0,), (0,)), ((), ())),
                               preferred_element_type=jnp.float32)


def _tgn_kernel(src_ref, dst_ref,
                nf_ref, ts_ref, ef_ref,
                srcc_ref, dstc_ref, srcr_ref, dstr_ref,
                Wnp_ref, bnp_ref,
                w0_ref, b0_ref, tw_ref, tb_ref,
                Wmsg_ref, bmsg_ref,
                Wih_ref, bih_ref,
                Whh_ref, bhh_ref,
                Wgate_ref, bgate_ref,
                Wproj_ref, bproj_ref,
                out_ref,
                mem_ref, econst_ref, T_ref):
    mem_ref[...] = _dg(nf_ref[...], Wnp_ref[...]) + bnp_ref[...]

    if True:
        t = ts_ref[...]
        lin = t * w0_ref[0, 0] + b0_ref[0, 0]
        sn = jnp.sin(t * tw_ref[...] + tb_ref[...])
        tf = jnp.concatenate([lin, sn], axis=1)
        Wmsg = Wmsg_ref[...]
        W_e = Wmsg[:, 2 * D:2 * D + EF]
        W_t = Wmsg[:, 2 * D + EF:]
        econst_ref[...] = (_dg(ef_ref[...], W_e) + _dg(tf, W_t)
                           + bmsg_ref[...])

        W_sd = Wmsg[:, :2 * D]
        Wih = Wih_ref[...]
        bih = bih_ref[...]
        Whh = Whh_ref[...]
        bhh = bhh_ref[...]
        econst = econst_ref[...]

        def init_body(e, carry):
            s = src_ref[e]
            d = dst_ref[e]
            T_ref[pl.ds(e, 1), :] = mem_ref[pl.ds(s, 1), :]
            T_ref[pl.ds(E + e, 1), :] = mem_ref[pl.ds(d, 1), :]
            return carry

        jax.lax.fori_loop(0, E, init_body, 0, unroll=8)

        src_c = srcc_ref[...]
        dst_c = dstc_ref[...]
        src_r = srcr_ref[...]
        dst_r = dstr_ref[...]
        all_r = jnp.concatenate([src_r, dst_r], axis=1)

        eqs = (src_c == all_r).astype(jnp.float32)
        eqd = (dst_c == all_r).astype(jnp.float32)
        bsm = eqs * (1.0 - eqd)
        eqsd = (src_c == dst_c)

        conf = ((src_c == src_r) | (src_c == dst_r)
                | (dst_c == src_r) | (dst_c == dst_r))
        row_i = jax.lax.broadcasted_iota(jnp.int32, (E, E), 0)
        col_i = jax.lax.broadcasted_iota(jnp.int32, (E, E), 1)
        lower = col_i < row_i
        CL = (conf & lower).astype(jnp.float32)
        ident = (row_i == col_i).astype(jnp.float32)
        ones8 = jnp.ones((E, 8), jnp.float32)

        def gru_combine(gi, gh, h):
            r = jax.nn.sigmoid(gi[:, :D] + gh[:, :D])
            z = jax.nn.sigmoid(gi[:, D:2 * D] + gh[:, D:2 * D])
            n = jnp.tanh(gi[:, 2 * D:] + r * gh[:, 2 * D:])
            return (1.0 - z) * n + z * h

        def wave_cond(carry):
            com_c, com_r = carry
            return jnp.sum(com_c) < jnp.float32(E)

        def wave_body(carry):
            com_c, com_r = carry
            blocked = jnp.max(CL * (1.0 - com_r), axis=1, keepdims=True)
            active = (1.0 - com_c) * (1.0 - blocked)

            Tv = T_ref[...]
            s_rows = Tv[:E, :]
            d_rows = Tv[E:, :]
            sd_flat = jnp.concatenate([s_rows, d_rows], axis=1)
            pre = _dg(sd_flat, W_sd) + econst
            msg = jnp.maximum(pre, 0.0)
            gh_all = _dg(Tv, Whh) + bhh
            gi = _dg(msg, Wih) + bih
            upd_s = gru_combine(gi, gh_all[:E, :], s_rows)
            gh_d2 = _dg(upd_s, Whh) + bhh
            gh_d = jnp.where(eqsd, gh_d2, gh_all[E:, :])
            h2 = jnp.where(eqsd, upd_s, d_rows)
            upd_d = gru_combine(gi, gh_d, h2)

            A_s = bsm * active
            A_d = eqd * active
            sc_s = _dgT(A_s, upd_s)
            sc_d = _dgT(A_d, upd_d)
            cov = _dgT(A_s + A_d, ones8)[:, :1]
            T_ref[...] = Tv * (1.0 - cov) + sc_s + sc_d

            com_c = com_c + active
            com8 = jnp.broadcast_to(com_c, (E, 8))
            com_r = _dgT(com8, ident)[:1, :]
            return com_c, com_r

        jax.lax.while_loop(
            wave_cond, wave_body,
            (jnp.zeros((E, 1), jnp.float32), jnp.zeros((1, E), jnp.float32)))

        def fin_body(e, carry):
            s = src_ref[e]
            d = dst_ref[e]
            mem_ref[pl.ds(s, 1), :] = T_ref[pl.ds(e, 1), :]
            mem_ref[pl.ds(d, 1), :] = T_ref[pl.ds(E + e, 1), :]
            return carry

        jax.lax.fori_loop(0, E, fin_body, 0, unroll=8)

        Wgate = Wgate_ref[...]
        bgate = bgate_ref[0, 0]
        Wproj = Wproj_ref[...]

        def ro_body(k, carry):
            m, zz, acc = carry
            tile = mem_ref[pl.ds(k * TILE, TILE), :]
            g = jnp.sum(tile * Wgate, axis=1, keepdims=True) + bgate
            mt = jnp.maximum(m, jnp.max(g))
            scale = jnp.exp(m - mt)
            w = jnp.exp(g - mt)
            p = _dg(tile, Wproj)
            acc = acc * scale + jnp.sum(w * p, axis=0, keepdims=True)
            zz = zz * scale + jnp.sum(w)
            return mt, zz, acc

        m, zz, acc = jax.lax.fori_loop(
            0, NT, ro_body,
            (jnp.float32(-jnp.inf), jnp.float32(0.0),
             jnp.zeros((1, D), jnp.float32)))
        out_ref[...] = acc / zz + bproj_ref[...]


@functools.partial(jax.jit, static_argnames=("interpret",))
def kernel(node_features, timestamps, edge_features, W_np, b_np, t2v_w0,
           t2v_b0, t2v_w, t2v_b, W_msg, b_msg, W_ih, b_ih, W_hh, b_hh,
           W_gate, b_gate, W_proj, b_proj, sources, destinations,
           interpret=False):
    src = sources.astype(jnp.int32)
    dst = destinations.astype(jnp.int32)
    ts = timestamps.reshape(E, 1).astype(jnp.float32)
    src_col = src.reshape(E, 1)
    dst_col = dst.reshape(E, 1)
    src_row = src.reshape(1, E)
    dst_row = dst.reshape(1, E)

    smem = lambda: pl.BlockSpec(memory_space=pltpu.SMEM)
    vfull = lambda: pl.BlockSpec(memory_space=pltpu.VMEM)

    grid_spec = pltpu.PrefetchScalarGridSpec(
        num_scalar_prefetch=2,
        grid=(1,),
        in_specs=[
            vfull(),
            vfull(),
            vfull(),
            vfull(), vfull(), vfull(), vfull(),
            vfull(), vfull(),
            smem(), smem(),
            vfull(), vfull(),
            vfull(), vfull(),
            vfull(), vfull(),
            vfull(), vfull(),
            vfull(), smem(),
            vfull(), vfull(),
        ],
        out_specs=pl.BlockSpec((1, D), lambda i, *_: (0, 0)),
        scratch_shapes=[
            pltpu.VMEM((N, D), jnp.float32),
            pltpu.VMEM((E, D), jnp.float32),
            pltpu.VMEM((2 * E, D), jnp.float32),
        ],
    )

    pooled = pl.pallas_call(
        _tgn_kernel,
        grid_spec=grid_spec,
        out_shape=jax.ShapeDtypeStruct((1, D), jnp.float32),
        compiler_params=pltpu.CompilerParams(
            dimension_semantics=("arbitrary",)),
        interpret=interpret,
    )(src, dst,
      node_features, ts, edge_features,
      src_col, dst_col, src_row, dst_row,
      W_np, b_np.reshape(1, D),
      t2v_w0.reshape(1, 1), t2v_b0.reshape(1, 1),
      t2v_w.reshape(1, TD - 1), t2v_b.reshape(1, TD - 1),
      W_msg, b_msg.reshape(1, D),
      W_ih, b_ih.reshape(1, 3 * D),
      W_hh, b_hh.reshape(1, 3 * D),
      W_gate, b_gate.reshape(1, 1),
      W_proj, b_proj.reshape(1, D))
    return pooled.reshape(D)


# speedup vs baseline: 1.6584x; 1.2961x over previous
import functools

import jax
import jax.numpy as jnp
from jax.experimental import pallas as pl
from jax.experimental.pallas import tpu as pltpu

N = 10000
E = 200
NF = 128
EF = 30
D = 128
TD = 16

NT = 10
TILE = N // NT


def _dg(a, b):
    return jax.lax.dot_general(a, b, (((1,), (1,)), ((), ())),
                               preferred_element_type=jnp.float32)


def _dgT(a, b):
    return jax.lax.dot_general(a, b, (((0,), (0,)), ((), ())),
                               preferred_element_type=jnp.float32)


def _tgn_kernel(src_ref, dst_ref,
                nf_ref, ts_ref, ef_ref,
                Wnp_ref, bnp_ref,
                w0_ref, b0_ref, tw_ref, tb_ref,
                Wmsg_ref, bmsg_ref,
                Wih_ref, bih_ref,
                Whh_ref, bhh_ref,
                Wgate_ref, bgate_ref,
                Wproj_ref, bproj_ref,
                out_ref,
                mem_ref, econst_ref, T_ref, idc_ref):
    mem_ref[...] = _dg(nf_ref[...], Wnp_ref[...]) + bnp_ref[...].reshape(1, D)

    if True:
        row_i = jax.lax.broadcasted_iota(jnp.int32, (E, E), 0)
        col_i = jax.lax.broadcasted_iota(jnp.int32, (E, E), 1)
        ident = (row_i == col_i).astype(jnp.float32)

        ts_row = ts_ref[...].reshape(1, E)
        ts8 = jnp.broadcast_to(ts_row, (8, E))
        t = jax.lax.dot_general(
            ident, ts8, (((1,), (1,)), ((), ())),
            preferred_element_type=jnp.float32,
            precision=jax.lax.Precision.HIGHEST)[:, :1]
        lin = t * w0_ref[0] + b0_ref[0]
        sn = jnp.sin(t * tw_ref[...].reshape(1, TD - 1)
                     + tb_ref[...].reshape(1, TD - 1))
        tf = jnp.concatenate([lin, sn], axis=1)
        Wmsg = Wmsg_ref[...]
        W_e = Wmsg[:, 2 * D:2 * D + EF]
        W_t = Wmsg[:, 2 * D + EF:]
        econst_ref[...] = (_dg(ef_ref[...], W_e) + _dg(tf, W_t)
                           + bmsg_ref[...].reshape(1, D))

        W_sd = Wmsg[:, :2 * D]
        Wih = Wih_ref[...]
        bih = bih_ref[...].reshape(1, 3 * D)
        Whh = Whh_ref[...]
        bhh = bhh_ref[...].reshape(1, 3 * D)
        econst = econst_ref[...]

        def init_body(e, carry):
            s = src_ref[e]
            d = dst_ref[e]
            T_ref[pl.ds(e, 1), :] = mem_ref[pl.ds(s, 1), :]
            T_ref[pl.ds(E + e, 1), :] = mem_ref[pl.ds(d, 1), :]
            idc_ref[pl.ds(e, 1), 0:1] = jnp.full((1, 1), s, jnp.float32)
            idc_ref[pl.ds(e, 1), 1:2] = jnp.full((1, 1), d, jnp.float32)
            return carry

        jax.lax.fori_loop(0, E, init_body, 0, unroll=8)

        src_c = idc_ref[:, 0:1]
        dst_c = idc_ref[:, 1:2]
        idT = jax.lax.dot_general(
            jnp.broadcast_to(idc_ref[...], (E, 2)), ident,
            (((0,), (0,)), ((), ())),
            preferred_element_type=jnp.float32,
            precision=jax.lax.Precision.HIGHEST)
        src_r = idT[0:1, :]
        dst_r = idT[1:2, :]
        all_r = jnp.concatenate([src_r, dst_r], axis=1)

        eqs = (src_c == all_r).astype(jnp.float32)
        eqd = (dst_c == all_r).astype(jnp.float32)
        bsm = eqs * (1.0 - eqd)
        eqsd = (src_c == dst_c)

        conf = ((src_c == src_r) | (src_c == dst_r)
                | (dst_c == src_r) | (dst_c == dst_r))
        lower = col_i < row_i
        CL = (conf & lower).astype(jnp.float32)
        ones8 = jnp.ones((E, 8), jnp.float32)

        def gru_combine(gi, gh, h):
            r = jax.nn.sigmoid(gi[:, :D] + gh[:, :D])
            z = jax.nn.sigmoid(gi[:, D:2 * D] + gh[:, D:2 * D])
            n = jnp.tanh(gi[:, 2 * D:] + r * gh[:, 2 * D:])
            return (1.0 - z) * n + z * h

        def wave_cond(carry):
            com_c, com_r = carry
            return jnp.sum(com_c) < jnp.float32(E)

        def wave_body(carry):
            com_c, com_r = carry
            blocked = jnp.max(CL * (1.0 - com_r), axis=1, keepdims=True)
            active = (1.0 - com_c) * (1.0 - blocked)

            Tv = T_ref[...]
            s_rows = Tv[:E, :]
            d_rows = Tv[E:, :]
            sd_flat = jnp.concatenate([s_rows, d_rows], axis=1)
            pre = _dg(sd_flat, W_sd) + econst
            msg = jnp.maximum(pre, 0.0)
            gh_all = _dg(Tv, Whh) + bhh
            gi = _dg(msg, Wih) + bih
            upd_s = gru_combine(gi, gh_all[:E, :], s_rows)
            gh_d2 = _dg(upd_s, Whh) + bhh
            gh_d = jnp.where(eqsd, gh_d2, gh_all[E:, :])
            h2 = jnp.where(eqsd, upd_s, d_rows)
            upd_d = gru_combine(gi, gh_d, h2)

            A_s = bsm * active
            A_d = eqd * active
            sc_s = _dgT(A_s, upd_s)
            sc_d = _dgT(A_d, upd_d)
            cov = _dgT(A_s + A_d, ones8)[:, :1]
            T_ref[...] = Tv * (1.0 - cov) + sc_s + sc_d

            com_c = com_c + active
            com8 = jnp.broadcast_to(com_c, (E, 8))
            com_r = _dgT(com8, ident)[:1, :]
            return com_c, com_r

        jax.lax.while_loop(
            wave_cond, wave_body,
            (jnp.zeros((E, 1), jnp.float32), jnp.zeros((1, E), jnp.float32)))

        def fin_body(e, carry):
            s = src_ref[e]
            d = dst_ref[e]
            mem_ref[pl.ds(s, 1), :] = T_ref[pl.ds(e, 1), :]
            mem_ref[pl.ds(d, 1), :] = T_ref[pl.ds(E + e, 1), :]
            return carry

        jax.lax.fori_loop(0, E, fin_body, 0, unroll=8)

        Wgate = Wgate_ref[...]
        bgate = bgate_ref[0]
        Wproj = Wproj_ref[...]

        def ro_body(k, carry):
            m, zz, acc = carry
            tile = mem_ref[pl.ds(k * TILE, TILE), :]
            g = jnp.sum(tile * Wgate, axis=1, keepdims=True) + bgate
            mt = jnp.maximum(m, jnp.max(g))
            scale = jnp.exp(m - mt)
            w = jnp.exp(g - mt)
            p = _dg(tile, Wproj)
            acc = acc * scale + jnp.sum(w * p, axis=0, keepdims=True)
            zz = zz * scale + jnp.sum(w)
            return mt, zz, acc

        m, zz, acc = jax.lax.fori_loop(
            0, NT, ro_body,
            (jnp.float32(-jnp.inf), jnp.float32(0.0),
             jnp.zeros((1, D), jnp.float32)))
        out_ref[...] = acc / zz + bproj_ref[...].reshape(1, D)


@functools.partial(jax.jit, static_argnames=("interpret",))
def kernel(node_features, timestamps, edge_features, W_np, b_np, t2v_w0,
           t2v_b0, t2v_w, t2v_b, W_msg, b_msg, W_ih, b_ih, W_hh, b_hh,
           W_gate, b_gate, W_proj, b_proj, sources, destinations,
           interpret=False):
    src = sources.astype(jnp.int32)
    dst = destinations.astype(jnp.int32)

    smem = lambda: pl.BlockSpec(memory_space=pltpu.SMEM)
    vfull = lambda: pl.BlockSpec(memory_space=pltpu.VMEM)

    grid_spec = pltpu.PrefetchScalarGridSpec(
        num_scalar_prefetch=2,
        grid=(1,),
        in_specs=[
            vfull(),
            vfull(),
            vfull(),
            vfull(), vfull(),
            smem(), smem(),
            vfull(), vfull(),
            vfull(), vfull(),
            vfull(), vfull(),
            vfull(), vfull(),
            vfull(), smem(),
            vfull(), vfull(),
        ],
        out_specs=pl.BlockSpec((1, D), lambda i, *_: (0, 0)),
        scratch_shapes=[
            pltpu.VMEM((N, D), jnp.float32),
            pltpu.VMEM((E, D), jnp.float32),
            pltpu.VMEM((2 * E, D), jnp.float32),
            pltpu.VMEM((E, 2), jnp.float32),
        ],
    )

    pooled = pl.pallas_call(
        _tgn_kernel,
        grid_spec=grid_spec,
        out_shape=jax.ShapeDtypeStruct((1, D), jnp.float32),
        compiler_params=pltpu.CompilerParams(
            dimension_semantics=("arbitrary",)),
        interpret=interpret,
    )(src, dst,
      node_features, timestamps, edge_features,
      W_np, b_np,
      t2v_w0, t2v_b0,
      t2v_w, t2v_b,
      W_msg, b_msg,
      W_ih, b_ih,
      W_hh, b_hh,
      W_gate, b_gate,
      W_proj, b_proj)
    return pooled.reshape(D)


# speedup vs baseline: 1.6744x; 1.0096x over previous
import functools

import jax
import jax.numpy as jnp
from jax.experimental import pallas as pl
from jax.experimental.pallas import tpu as pltpu

N = 10000
E = 200
NF = 128
EF = 30
D = 128
TD = 16

NT = 10
TILE = N // NT


def _dg(a, b):
    return jax.lax.dot_general(a, b, (((1,), (1,)), ((), ())),
                               preferred_element_type=jnp.float32)


def _dgT(a, b):
    return jax.lax.dot_general(a, b, (((0,), (0,)), ((), ())),
                               preferred_element_type=jnp.float32)


def _tgn_kernel(src_ref, dst_ref,
                nf_ref, ts_ref, ef_ref,
                Wnp_ref, bnp_ref,
                w0_ref, b0_ref, tw_ref, tb_ref,
                Wmsg_ref, bmsg_ref,
                Wih_ref, bih_ref,
                Whh_ref, bhh_ref,
                Wgate_ref, bgate_ref,
                Wproj_ref, bproj_ref,
                out_ref,
                mem_ref, econst_ref, T_ref, idc_ref, nfv_ref, dma_sem):
    nf_copy = pltpu.make_async_copy(nf_ref, nfv_ref, dma_sem)
    nf_copy.start()

    if True:
        row_i = jax.lax.broadcasted_iota(jnp.int32, (E, E), 0)
        col_i = jax.lax.broadcasted_iota(jnp.int32, (E, E), 1)
        ident = (row_i == col_i).astype(jnp.float32)

        ts_row = ts_ref[...].reshape(1, E)
        ts8 = jnp.broadcast_to(ts_row, (8, E))
        t = jax.lax.dot_general(
            ident, ts8, (((1,), (1,)), ((), ())),
            preferred_element_type=jnp.float32,
            precision=jax.lax.Precision.HIGHEST)[:, :1]
        lin = t * w0_ref[0] + b0_ref[0]
        sn = jnp.sin(t * tw_ref[...].reshape(1, TD - 1)
                     + tb_ref[...].reshape(1, TD - 1))
        tf = jnp.concatenate([lin, sn], axis=1)
        Wmsg = Wmsg_ref[...]
        W_e = Wmsg[:, 2 * D:2 * D + EF]
        W_t = Wmsg[:, 2 * D + EF:]
        econst_ref[...] = (_dg(ef_ref[...], W_e) + _dg(tf, W_t)
                           + bmsg_ref[...].reshape(1, D))

        W_sd = Wmsg[:, :2 * D]
        Wih = Wih_ref[...]
        bih = bih_ref[...].reshape(1, 3 * D)
        Whh = Whh_ref[...]
        bhh = bhh_ref[...].reshape(1, 3 * D)
        econst = econst_ref[...]

        def id_body(e, carry):
            idc_ref[pl.ds(e, 1), 0:1] = jnp.full((1, 1), src_ref[e], jnp.float32)
            idc_ref[pl.ds(e, 1), 1:2] = jnp.full((1, 1), dst_ref[e], jnp.float32)
            return carry

        jax.lax.fori_loop(0, E, id_body, 0, unroll=8)

        src_c = idc_ref[:, 0:1]
        dst_c = idc_ref[:, 1:2]
        idT = jax.lax.dot_general(
            jnp.broadcast_to(idc_ref[...], (E, 2)), ident,
            (((0,), (0,)), ((), ())),
            preferred_element_type=jnp.float32,
            precision=jax.lax.Precision.HIGHEST)
        src_r = idT[0:1, :]
        dst_r = idT[1:2, :]
        all_r = jnp.concatenate([src_r, dst_r], axis=1)

        eqs = (src_c == all_r).astype(jnp.float32)
        eqd = (dst_c == all_r).astype(jnp.float32)
        bsm = eqs * (1.0 - eqd)
        eqsd = (src_c == dst_c)

        conf = ((src_c == src_r) | (src_c == dst_r)
                | (dst_c == src_r) | (dst_c == dst_r))
        lower = col_i < row_i
        CL = (conf & lower).astype(jnp.float32)
        ones8 = jnp.ones((E, 8), jnp.float32)

        nf_copy.wait()
        mem_ref[...] = _dg(nfv_ref[...], Wnp_ref[...]) + bnp_ref[...].reshape(1, D)

        def init_body(e, carry):
            s = src_ref[e]
            d = dst_ref[e]
            T_ref[pl.ds(e, 1), :] = mem_ref[pl.ds(s, 1), :]
            T_ref[pl.ds(E + e, 1), :] = mem_ref[pl.ds(d, 1), :]
            return carry

        jax.lax.fori_loop(0, E, init_body, 0, unroll=8)


        def gru_combine(gi, gh, h):
            r = jax.nn.sigmoid(gi[:, :D] + gh[:, :D])
            z = jax.nn.sigmoid(gi[:, D:2 * D] + gh[:, D:2 * D])
            n = jnp.tanh(gi[:, 2 * D:] + r * gh[:, 2 * D:])
            return (1.0 - z) * n + z * h

        def wave_cond(carry):
            com_c, com_r = carry
            return jnp.sum(com_c) < jnp.float32(E)

        def wave_body(carry):
            com_c, com_r = carry
            blocked = jnp.max(CL * (1.0 - com_r), axis=1, keepdims=True)
            active = (1.0 - com_c) * (1.0 - blocked)

            Tv = T_ref[...]
            s_rows = Tv[:E, :]
            d_rows = Tv[E:, :]
            sd_flat = jnp.concatenate([s_rows, d_rows], axis=1)
            pre = _dg(sd_flat, W_sd) + econst
            msg = jnp.maximum(pre, 0.0)
            gh_all = _dg(Tv, Whh) + bhh
            gi = _dg(msg, Wih) + bih
            upd_s = gru_combine(gi, gh_all[:E, :], s_rows)
            gh_d2 = _dg(upd_s, Whh) + bhh
            gh_d = jnp.where(eqsd, gh_d2, gh_all[E:, :])
            h2 = jnp.where(eqsd, upd_s, d_rows)
            upd_d = gru_combine(gi, gh_d, h2)

            A_s = bsm * active
            A_d = eqd * active
            sc_s = _dgT(A_s, upd_s)
            sc_d = _dgT(A_d, upd_d)
            cov = _dgT(A_s + A_d, ones8)[:, :1]
            T_ref[...] = Tv * (1.0 - cov) + sc_s + sc_d

            com_c = com_c + active
            com8 = jnp.broadcast_to(com_c, (E, 8))
            com_r = _dgT(com8, ident)[:1, :]
            return com_c, com_r

        jax.lax.while_loop(
            wave_cond, wave_body,
            (jnp.zeros((E, 1), jnp.float32), jnp.zeros((1, E), jnp.float32)))

        def fin_body(e, carry):
            s = src_ref[e]
            d = dst_ref[e]
            mem_ref[pl.ds(s, 1), :] = T_ref[pl.ds(e, 1), :]
            mem_ref[pl.ds(d, 1), :] = T_ref[pl.ds(E + e, 1), :]
            return carry

        jax.lax.fori_loop(0, E, fin_body, 0, unroll=8)

        Wgate = Wgate_ref[...]
        bgate = bgate_ref[0]
        Wproj = Wproj_ref[...]

        def ro_body(k, carry):
            m, zz, acc = carry
            tile = mem_ref[pl.ds(k * TILE, TILE), :]
            g = jnp.sum(tile * Wgate, axis=1, keepdims=True) + bgate
            mt = jnp.maximum(m, jnp.max(g))
            scale = jnp.exp(m - mt)
            w = jnp.exp(g - mt)
            p = _dg(tile, Wproj)
            acc = acc * scale + jnp.sum(w * p, axis=0, keepdims=True)
            zz = zz * scale + jnp.sum(w)
            return mt, zz, acc

        m, zz, acc = jax.lax.fori_loop(
            0, NT, ro_body,
            (jnp.float32(-jnp.inf), jnp.float32(0.0),
             jnp.zeros((1, D), jnp.float32)))
        out_ref[...] = acc / zz + bproj_ref[...].reshape(1, D)


@functools.partial(jax.jit, static_argnames=("interpret",))
def kernel(node_features, timestamps, edge_features, W_np, b_np, t2v_w0,
           t2v_b0, t2v_w, t2v_b, W_msg, b_msg, W_ih, b_ih, W_hh, b_hh,
           W_gate, b_gate, W_proj, b_proj, sources, destinations,
           interpret=False):
    src = sources.astype(jnp.int32)
    dst = destinations.astype(jnp.int32)

    smem = lambda: pl.BlockSpec(memory_space=pltpu.SMEM)
    vfull = lambda: pl.BlockSpec(memory_space=pltpu.VMEM)

    grid_spec = pltpu.PrefetchScalarGridSpec(
        num_scalar_prefetch=2,
        grid=(1,),
        in_specs=[
            pl.BlockSpec(memory_space=pl.ANY),
            vfull(),
            vfull(),
            vfull(), vfull(),
            smem(), smem(),
            vfull(), vfull(),
            vfull(), vfull(),
            vfull(), vfull(),
            vfull(), vfull(),
            vfull(), smem(),
            vfull(), vfull(),
        ],
        out_specs=pl.BlockSpec((1, D), lambda i, *_: (0, 0)),
        scratch_shapes=[
            pltpu.VMEM((N, D), jnp.float32),
            pltpu.VMEM((E, D), jnp.float32),
            pltpu.VMEM((2 * E, D), jnp.float32),
            pltpu.VMEM((E, 2), jnp.float32),
            pltpu.VMEM((N, NF), jnp.float32),
            pltpu.SemaphoreType.DMA,
        ],
    )

    pooled = pl.pallas_call(
        _tgn_kernel,
        grid_spec=grid_spec,
        out_shape=jax.ShapeDtypeStruct((1, D), jnp.float32),
        compiler_params=pltpu.CompilerParams(
            dimension_semantics=("arbitrary",)),
        interpret=interpret,
    )(src, dst,
      node_features, timestamps, edge_features,
      W_np, b_np,
      t2v_w0, t2v_b0,
      t2v_w, t2v_b,
      W_msg, b_msg,
      W_ih, b_ih,
      W_hh, b_hh,
      W_gate, b_gate,
      W_proj, b_proj)
    return pooled.reshape(D)


# speedup vs baseline: 1.7813x; 1.0639x over previous
import functools

import jax
import jax.numpy as jnp
from jax.experimental import pallas as pl
from jax.experimental.pallas import tpu as pltpu

N = 10000
E = 200
NF = 128
EF = 30
D = 128
TD = 16

NT = 10
TILE = N // NT


def _dg(a, b):
    return jax.lax.dot_general(a, b, (((1,), (1,)), ((), ())),
                               preferred_element_type=jnp.float32)


def _dgT(a, b):
    return jax.lax.dot_general(a, b, (((0,), (0,)), ((), ())),
                               preferred_element_type=jnp.float32)


def _tgn_kernel(src_ref, dst_ref,
                nf_ref, ts_ref, ef_ref,
                Wnp_ref, bnp_ref,
                w0_ref, b0_ref, tw_ref, tb_ref,
                Wmsg_ref, bmsg_ref,
                Wih_ref, bih_ref,
                Whh_ref, bhh_ref,
                Wgate_ref, bgate_ref,
                Wproj_ref, bproj_ref,
                out_ref,
                mem_ref, econst_ref, T_ref, idc_ref, nfv_ref, dma_sem):
    nf_copy = pltpu.make_async_copy(nf_ref, nfv_ref, dma_sem)
    nf_copy.start()

    if True:
        row_i = jax.lax.broadcasted_iota(jnp.int32, (E, E), 0)
        col_i = jax.lax.broadcasted_iota(jnp.int32, (E, E), 1)
        ident = (row_i == col_i).astype(jnp.float32)

        ts_row = ts_ref[...].reshape(1, E)
        ts8 = jnp.broadcast_to(ts_row, (8, E))
        t = jax.lax.dot_general(
            ident, ts8, (((1,), (1,)), ((), ())),
            preferred_element_type=jnp.float32,
            precision=jax.lax.Precision.HIGHEST)[:, :1]
        lin = t * w0_ref[0] + b0_ref[0]
        sn = jnp.sin(t * tw_ref[...].reshape(1, TD - 1)
                     + tb_ref[...].reshape(1, TD - 1))
        tf = jnp.concatenate([lin, sn], axis=1)
        Wmsg = Wmsg_ref[...]
        W_e = Wmsg[:, 2 * D:2 * D + EF]
        W_t = Wmsg[:, 2 * D + EF:]
        econst_ref[...] = (_dg(ef_ref[...], W_e) + _dg(tf, W_t)
                           + bmsg_ref[...].reshape(1, D))

        W_sd = Wmsg[:, :2 * D]
        Wih = Wih_ref[...]
        bih = bih_ref[...].reshape(1, 3 * D)
        Whh = Whh_ref[...]
        bhh = bhh_ref[...].reshape(1, 3 * D)
        econst = econst_ref[...]

        def id_body(e, carry):
            idc_ref[pl.ds(e, 1), 0:1] = jnp.full((1, 1), src_ref[e], jnp.float32)
            idc_ref[pl.ds(e, 1), 1:2] = jnp.full((1, 1), dst_ref[e], jnp.float32)
            return carry

        jax.lax.fori_loop(0, E, id_body, 0, unroll=8)

        src_c = idc_ref[:, 0:1]
        dst_c = idc_ref[:, 1:2]
        idT = jax.lax.dot_general(
            jnp.broadcast_to(idc_ref[...], (E, 2)), ident,
            (((0,), (0,)), ((), ())),
            preferred_element_type=jnp.float32,
            precision=jax.lax.Precision.HIGHEST)
        src_r = idT[0:1, :]
        dst_r = idT[1:2, :]
        all_r = jnp.concatenate([src_r, dst_r], axis=1)

        eqs = (src_c == all_r).astype(jnp.float32)
        eqd = (dst_c == all_r).astype(jnp.float32)
        bsm = eqs * (1.0 - eqd)
        eqsd = (src_c == dst_c)

        conf = ((src_c == src_r) | (src_c == dst_r)
                | (dst_c == src_r) | (dst_c == dst_r))
        lower = col_i < row_i
        CL = (conf & lower).astype(jnp.float32)
        ones8 = jnp.ones((E, 8), jnp.float32)

        nf_copy.wait()
        mem_ref[...] = _dg(nfv_ref[...], Wnp_ref[...]) + bnp_ref[...].reshape(1, D)

        def init_body(e, carry):
            s = src_ref[e]
            d = dst_ref[e]
            T_ref[pl.ds(e, 1), :] = mem_ref[pl.ds(s, 1), :]
            T_ref[pl.ds(E + e, 1), :] = mem_ref[pl.ds(d, 1), :]
            return carry

        jax.lax.fori_loop(0, E, init_body, 0, unroll=8)


        def gru_combine(gi, gh, h):
            r = jax.nn.sigmoid(gi[:, :D] + gh[:, :D])
            z = jax.nn.sigmoid(gi[:, D:2 * D] + gh[:, D:2 * D])
            n = jnp.tanh(gi[:, 2 * D:] + r * gh[:, 2 * D:])
            return (1.0 - z) * n + z * h

        def wave_cond(carry):
            com_c, com_r = carry
            return jnp.sum(com_c) < jnp.float32(E)

        def wave_body(carry):
            com_c, com_r = carry
            blocked = jnp.max(CL * (1.0 - com_r), axis=1, keepdims=True)
            active = (1.0 - com_c) * (1.0 - blocked)

            Tv = T_ref[...]
            s_rows = Tv[:E, :]
            d_rows = Tv[E:, :]
            sd_flat = jnp.concatenate([s_rows, d_rows], axis=1)
            pre = _dg(sd_flat, W_sd) + econst
            msg = jnp.maximum(pre, 0.0)
            gh_all = _dg(Tv, Whh) + bhh
            gi = _dg(msg, Wih) + bih
            upd_s = gru_combine(gi, gh_all[:E, :], s_rows)
            gh_d2 = _dg(upd_s, Whh) + bhh
            gh_d = jnp.where(eqsd, gh_d2, gh_all[E:, :])
            h2 = jnp.where(eqsd, upd_s, d_rows)
            upd_d = gru_combine(gi, gh_d, h2)

            A_s = bsm * active
            A_d = eqd * active
            sc_s = _dgT(A_s, upd_s)
            sc_d = _dgT(A_d, upd_d)
            cov = _dgT(A_s + A_d, ones8)[:, :1]
            T_ref[...] = Tv * (1.0 - cov) + sc_s + sc_d

            com_c = com_c + active
            com8 = jnp.broadcast_to(com_c, (E, 8))
            com_r = _dgT(com8, ident)[:1, :]
            return com_c, com_r

        jax.lax.while_loop(
            wave_cond, wave_body,
            (jnp.zeros((E, 1), jnp.float32), jnp.zeros((1, E), jnp.float32)))

        def fin_body(e, carry):
            s = src_ref[e]
            d = dst_ref[e]
            mem_ref[pl.ds(s, 1), :] = T_ref[pl.ds(e, 1), :]
            mem_ref[pl.ds(d, 1), :] = T_ref[pl.ds(E + e, 1), :]
            return carry

        jax.lax.fori_loop(0, E, fin_body, 0, unroll=8)

        Wg8 = jnp.broadcast_to(Wgate_ref[...], (8, D))
        mem_all = mem_ref[...]
        g8 = _dg(mem_all, Wg8) + bgate_ref[0]
        m = jnp.max(g8[:, 0:1])
        e8 = jnp.exp(g8 - m)
        zz = jnp.sum(e8[:, 0:1])
        sums = jax.lax.dot_general(
            e8, mem_all, (((0,), (0,)), ((), ())),
            preferred_element_type=jnp.float32,
            precision=jax.lax.Precision.HIGHEST)
        pooled = _dg(sums[0:1, :], Wproj_ref[...]) / zz
        out_ref[...] = pooled + bproj_ref[...].reshape(1, D)


@functools.partial(jax.jit, static_argnames=("interpret",))
def kernel(node_features, timestamps, edge_features, W_np, b_np, t2v_w0,
           t2v_b0, t2v_w, t2v_b, W_msg, b_msg, W_ih, b_ih, W_hh, b_hh,
           W_gate, b_gate, W_proj, b_proj, sources, destinations,
           interpret=False):
    src = sources.astype(jnp.int32)
    dst = destinations.astype(jnp.int32)

    smem = lambda: pl.BlockSpec(memory_space=pltpu.SMEM)
    vfull = lambda: pl.BlockSpec(memory_space=pltpu.VMEM)

    grid_spec = pltpu.PrefetchScalarGridSpec(
        num_scalar_prefetch=2,
        grid=(1,),
        in_specs=[
            pl.BlockSpec(memory_space=pl.ANY),
            vfull(),
            vfull(),
            vfull(), vfull(),
            smem(), smem(),
            vfull(), vfull(),
            vfull(), vfull(),
            vfull(), vfull(),
            vfull(), vfull(),
            vfull(), smem(),
            vfull(), vfull(),
        ],
        out_specs=pl.BlockSpec((1, D), lambda i, *_: (0, 0)),
        scratch_shapes=[
            pltpu.VMEM((N, D), jnp.float32),
            pltpu.VMEM((E, D), jnp.float32),
            pltpu.VMEM((2 * E, D), jnp.float32),
            pltpu.VMEM((E, 2), jnp.float32),
            pltpu.VMEM((N, NF), jnp.float32),
            pltpu.SemaphoreType.DMA,
        ],
    )

    pooled = pl.pallas_call(
        _tgn_kernel,
        grid_spec=grid_spec,
        out_shape=jax.ShapeDtypeStruct((1, D), jnp.float32),
        compiler_params=pltpu.CompilerParams(
            dimension_semantics=("arbitrary",)),
        interpret=interpret,
    )(src, dst,
      node_features, timestamps, edge_features,
      W_np, b_np,
      t2v_w0, t2v_b0,
      t2v_w, t2v_b,
      W_msg, b_msg,
      W_ih, b_ih,
      W_hh, b_hh,
      W_gate, b_gate,
      W_proj, b_proj)
    return pooled.reshape(D)


# speedup vs baseline: 1.7814x; 1.0001x over previous
import functools

import jax
import jax.numpy as jnp
from jax.experimental import pallas as pl
from jax.experimental.pallas import tpu as pltpu

N = 10000
E = 200
NF = 128
EF = 30
D = 128
TD = 16

def _dg(a, b):
    return jax.lax.dot_general(a, b, (((1,), (1,)), ((), ())),
                               preferred_element_type=jnp.float32)


def _dgT(a, b):
    return jax.lax.dot_general(a, b, (((0,), (0,)), ((), ())),
                               preferred_element_type=jnp.float32)


def _tgn_kernel(src_ref, dst_ref,
                nf_ref, ts_ref, ef_ref,
                Wnp_ref, bnp_ref,
                w0_ref, b0_ref, tw_ref, tb_ref,
                Wmsg_ref, bmsg_ref,
                Wih_ref, bih_ref,
                Whh_ref, bhh_ref,
                Wgate_ref, bgate_ref,
                Wproj_ref, bproj_ref,
                out_ref,
                mem_ref, econst_ref, T_ref, idc_ref, nfv_ref, dma_sem):
    nf_copy = pltpu.make_async_copy(nf_ref, nfv_ref, dma_sem)
    nf_copy.start()

    if True:
        row_i = jax.lax.broadcasted_iota(jnp.int32, (E, E), 0)
        col_i = jax.lax.broadcasted_iota(jnp.int32, (E, E), 1)
        ident = (row_i == col_i).astype(jnp.float32)

        ts_row = ts_ref[...].reshape(1, E)
        ts8 = jnp.broadcast_to(ts_row, (8, E))
        t = jax.lax.dot_general(
            ident, ts8, (((1,), (1,)), ((), ())),
            preferred_element_type=jnp.float32,
            precision=jax.lax.Precision.HIGHEST)[:, :1]
        lin = t * w0_ref[0] + b0_ref[0]
        sn = jnp.sin(t * tw_ref[...].reshape(1, TD - 1)
                     + tb_ref[...].reshape(1, TD - 1))
        tf = jnp.concatenate([lin, sn], axis=1)
        Wmsg = Wmsg_ref[...]
        W_e = Wmsg[:, 2 * D:2 * D + EF]
        W_t = Wmsg[:, 2 * D + EF:]
        econst_ref[...] = (_dg(ef_ref[...], W_e) + _dg(tf, W_t)
                           + bmsg_ref[...].reshape(1, D))

        W_sd = Wmsg[:, :2 * D]
        Wih = Wih_ref[...]
        bih = bih_ref[...].reshape(1, 3 * D)
        Whh = Whh_ref[...]
        bhh = bhh_ref[...].reshape(1, 3 * D)
        econst = econst_ref[...]

        def id_body(e, carry):
            idc_ref[pl.ds(e, 1), 0:1] = jnp.full((1, 1), src_ref[e], jnp.float32)
            idc_ref[pl.ds(e, 1), 1:2] = jnp.full((1, 1), dst_ref[e], jnp.float32)
            return carry

        jax.lax.fori_loop(0, E, id_body, 0, unroll=8)

        src_c = idc_ref[:, 0:1]
        dst_c = idc_ref[:, 1:2]
        idT = jax.lax.dot_general(
            jnp.broadcast_to(idc_ref[...], (E, 2)), ident,
            (((0,), (0,)), ((), ())),
            preferred_element_type=jnp.float32,
            precision=jax.lax.Precision.HIGHEST)
        src_r = idT[0:1, :]
        dst_r = idT[1:2, :]
        all_r = jnp.concatenate([src_r, dst_r], axis=1)

        eqs = (src_c == all_r).astype(jnp.float32)
        eqd = (dst_c == all_r).astype(jnp.float32)
        bsm = eqs * (1.0 - eqd)
        eqsd = (src_c == dst_c)

        conf = ((src_c == src_r) | (src_c == dst_r)
                | (dst_c == src_r) | (dst_c == dst_r))
        lower = col_i < row_i
        CL = (conf & lower).astype(jnp.float32)
        ones8 = jnp.ones((E, 8), jnp.float32)

        nf_copy.wait()
        mem_ref[...] = _dg(nfv_ref[...], Wnp_ref[...]) + bnp_ref[...].reshape(1, D)

        def init_body(e, carry):
            s = src_ref[e]
            d = dst_ref[e]
            T_ref[pl.ds(e, 1), :] = mem_ref[pl.ds(s, 1), :]
            T_ref[pl.ds(E + e, 1), :] = mem_ref[pl.ds(d, 1), :]
            return carry

        jax.lax.fori_loop(0, E, init_body, 0, unroll=8)


        def gru_combine(gi, gh, h):
            r = jax.nn.sigmoid(gi[:, :D] + gh[:, :D])
            z = jax.nn.sigmoid(gi[:, D:2 * D] + gh[:, D:2 * D])
            n = jnp.tanh(gi[:, 2 * D:] + r * gh[:, 2 * D:])
            return (1.0 - z) * n + z * h

        def wave_cond(carry):
            com_c, com_r = carry
            return jnp.sum(com_c) < jnp.float32(E)

        def wave_body(carry):
            com_c, com_r = carry
            blocked = jnp.max(CL * (1.0 - com_r), axis=1, keepdims=True)
            active = (1.0 - com_c) * (1.0 - blocked)

            Tv = T_ref[...]
            s_rows = Tv[:E, :]
            d_rows = Tv[E:, :]
            sd_flat = jnp.concatenate([s_rows, d_rows], axis=1)
            pre = _dg(sd_flat, W_sd) + econst
            msg = jnp.maximum(pre, 0.0)
            gh_all = _dg(Tv, Whh) + bhh
            gi = _dg(msg, Wih) + bih
            upd_s = gru_combine(gi, gh_all[:E, :], s_rows)
            gh_d2 = _dg(upd_s, Whh) + bhh
            gh_d = jnp.where(eqsd, gh_d2, gh_all[E:, :])
            h2 = jnp.where(eqsd, upd_s, d_rows)
            upd_d = gru_combine(gi, gh_d, h2)

            A_s = bsm * active
            A_d = eqd * active
            sc_s = _dgT(A_s, upd_s)
            sc_d = _dgT(A_d, upd_d)
            cov = _dgT(A_s + A_d, ones8)[:, :1]
            T_ref[...] = Tv * (1.0 - cov) + sc_s + sc_d

            com_c = com_c + active
            com8 = jnp.broadcast_to(com_c, (E, 8))
            com_r = _dgT(com8, ident)[:1, :]
            return com_c, com_r

        jax.lax.while_loop(
            wave_cond, wave_body,
            (jnp.zeros((E, 1), jnp.float32), jnp.zeros((1, E), jnp.float32)))

        def fin_body(e, carry):
            s = src_ref[e]
            d = dst_ref[e]
            mem_ref[pl.ds(s, 1), :] = T_ref[pl.ds(e, 1), :]
            mem_ref[pl.ds(d, 1), :] = T_ref[pl.ds(E + e, 1), :]
            return carry

        jax.lax.fori_loop(0, E, fin_body, 0, unroll=8)

        Wg8 = jnp.broadcast_to(Wgate_ref[...], (8, D))
        mem_all = mem_ref[...]
        g8 = _dg(mem_all, Wg8) + bgate_ref[0]
        m = jnp.max(g8[:, 0:1])
        e8 = jnp.exp(g8 - m)
        zz = jnp.sum(e8[:, 0:1])
        sums = jax.lax.dot_general(
            e8, mem_all, (((0,), (0,)), ((), ())),
            preferred_element_type=jnp.float32,
            precision=jax.lax.Precision.HIGHEST)
        pooled = _dg(sums[0:1, :], Wproj_ref[...]) / zz
        out_ref[...] = pooled + bproj_ref[...].reshape(1, D)


@functools.partial(jax.jit, static_argnames=("interpret",))
def kernel(node_features, timestamps, edge_features, W_np, b_np, t2v_w0,
           t2v_b0, t2v_w, t2v_b, W_msg, b_msg, W_ih, b_ih, W_hh, b_hh,
           W_gate, b_gate, W_proj, b_proj, sources, destinations,
           interpret=False):
    src = sources.astype(jnp.int32)
    dst = destinations.astype(jnp.int32)

    smem = lambda: pl.BlockSpec(memory_space=pltpu.SMEM)
    vfull = lambda: pl.BlockSpec(memory_space=pltpu.VMEM)

    grid_spec = pltpu.PrefetchScalarGridSpec(
        num_scalar_prefetch=2,
        grid=(1,),
        in_specs=[
            pl.BlockSpec(memory_space=pl.ANY),
            vfull(),
            vfull(),
            vfull(), vfull(),
            smem(), smem(),
            vfull(), vfull(),
            vfull(), vfull(),
            vfull(), vfull(),
            vfull(), vfull(),
            vfull(), smem(),
            vfull(), vfull(),
        ],
        out_specs=pl.BlockSpec((1, D), lambda i, *_: (0, 0)),
        scratch_shapes=[
            pltpu.VMEM((N, D), jnp.float32),
            pltpu.VMEM((E, D), jnp.float32),
            pltpu.VMEM((2 * E, D), jnp.float32),
            pltpu.VMEM((E, 2), jnp.float32),
            pltpu.VMEM((N, NF), jnp.float32),
            pltpu.SemaphoreType.DMA,
        ],
    )

    pooled = pl.pallas_call(
        _tgn_kernel,
        grid_spec=grid_spec,
        out_shape=jax.ShapeDtypeStruct((1, D), jnp.float32),
        compiler_params=pltpu.CompilerParams(
            dimension_semantics=("arbitrary",)),
        interpret=interpret,
    )(src, dst,
      node_features, timestamps, edge_features,
      W_np, b_np,
      t2v_w0, t2v_b0,
      t2v_w, t2v_b,
      W_msg, b_msg,
      W_ih, b_ih,
      W_hh, b_hh,
      W_gate, b_gate,
      W_proj, b_proj)
    return pooled.reshape(D)
